# Initial kernel scaffold; baseline (speedup 1.0000x reference)
import jax
import jax.numpy as jnp
from jax.experimental import pallas as pl


def kernel(image, mask, W, b):
    raise NotImplementedError("write your pallas kernel here")


# speedup vs baseline: 1.3392x; 1.3392x over previous
import jax
import jax.numpy as jnp
from jax.experimental import pallas as pl


def _body(img_ref, msk_ref, w_ref, b_ref, out_ref):
    x = img_ref[0]
    pooled = jnp.mean(x, axis=(1, 2))
    logits = jnp.sum(pooled[:, None] * w_ref[...], axis=0) + b_ref[...]
    sig = jax.nn.sigmoid(logits)
    color = sig[:3]
    t = sig[3]
    cb = (color * (1.0 - t))[:, None, None]
    m = (msk_ref[0] != 0)[None, :, :]
    out_ref[0] = jnp.where(m, cb + t * x, x)


def kernel(image, mask, W, b):
    B, C, H, Wd = image.shape
    return pl.pallas_call(
        _body,
        grid=(B,),
        in_specs=[
            pl.BlockSpec((1, C, H, Wd), lambda i: (i, 0, 0, 0)),
            pl.BlockSpec((1, H, Wd), lambda i: (i, 0, 0)),
            pl.BlockSpec((C, 4), lambda i: (0, 0)),
            pl.BlockSpec((4,), lambda i: (0,)),
        ],
        out_specs=pl.BlockSpec((1, C, H, Wd), lambda i: (i, 0, 0, 0)),
        out_shape=jax.ShapeDtypeStruct(image.shape, image.dtype),
    )(image, mask, W, b)


# speedup vs baseline: 1.3433x; 1.0030x over previous
import jax
import jax.numpy as jnp
from jax.experimental import pallas as pl
from jax.experimental.pallas import tpu as pltpu


def _body(img_ref, msk_ref, w_ref, b_ref, out_ref):
    x = img_ref[0]
    pooled = jnp.mean(x, axis=(1, 2))
    logits = jnp.sum(pooled[:, None] * w_ref[...], axis=0) + b_ref[...]
    sig = jax.nn.sigmoid(logits)
    color = sig[:3]
    t = sig[3]
    cb = (color * (1.0 - t))[:, None, None]
    m = (msk_ref[0] != 0)[None, :, :]
    out_ref[0] = jnp.where(m, cb + t * x, x)


def kernel(image, mask, W, b):
    B, C, H, Wd = image.shape
    return pl.pallas_call(
        _body,
        grid=(B,),
        in_specs=[
            pl.BlockSpec((1, C, H, Wd), lambda i: (i, 0, 0, 0)),
            pl.BlockSpec((1, H, Wd), lambda i: (i, 0, 0)),
            pl.BlockSpec((C, 4), lambda i: (0, 0)),
            pl.BlockSpec((4,), lambda i: (0,)),
        ],
        out_specs=pl.BlockSpec((1, C, H, Wd), lambda i: (i, 0, 0, 0)),
        out_shape=jax.ShapeDtypeStruct(image.shape, image.dtype),
        compiler_params=pltpu.CompilerParams(
            dimension_semantics=("arbitrary",),
            vmem_limit_bytes=100 * 1024 * 1024,
        ),
    )(image, mask, W, b)
